# baseline (speedup 1.0000x reference)
import functools
import jax
import jax.numpy as jnp
from jax import lax
from jax.experimental import pallas as pl
from jax.experimental.pallas import tpu as pltpu
from jax.experimental.pallas import tpu_sc as plsc

N = 10000
HID = 128
OUT = 64
HALF = HID // 2
NC, NS, LANES = 2, 16, 16
STRIPE = 640
NPAD = NS * STRIPE
CHUNK = 128
CPT = 160
EPT = CHUNK * CPT
EPAD = EPT * NS
DEG_CPT = CPT // NC


@functools.lru_cache(maxsize=None)
def _sc_kernels():
  mesh = plsc.VectorSubcoreMesh(core_axis_name="c", subcore_axis_name="s",
                                num_cores=NC, num_subcores=NS)
  params = pltpu.CompilerParams(use_tc_tiling_on_sc=False,
                                needs_layout_passes=False)

  @functools.partial(
      pl.kernel,
      out_type=jax.ShapeDtypeStruct((NC, NPAD), jnp.float32),
      mesh=mesh,
      compiler_params=params,
      scratch_types=[
          pltpu.VMEM((DEG_CPT, CHUNK), jnp.int32),
          pltpu.VMEM((DEG_CPT, CHUNK), jnp.float32),
          pltpu.VMEM((STRIPE,), jnp.float32),
          pltpu.VMEM_SHARED((NPAD,), jnp.float32),
      ],
  )
  def deg_kernel(col_hbm, ew_hbm, out_hbm, col_v, ew_v, zeros_v, deg_sh):
    c = lax.axis_index("c")
    s = lax.axis_index("s")
    pltpu.sync_copy(col_hbm.at[s, pl.ds(c * DEG_CPT, DEG_CPT)], col_v)
    pltpu.sync_copy(ew_hbm.at[s, pl.ds(c * DEG_CPT, DEG_CPT)], ew_v)

    @pl.loop(0, STRIPE // LANES)
    def _(j):
      zeros_v[pl.ds(j * LANES, LANES)] = jnp.zeros((LANES,), jnp.float32)

    pltpu.sync_copy(zeros_v, deg_sh.at[pl.ds(s * STRIPE, STRIPE)])
    plsc.subcore_barrier()

    @pl.loop(0, DEG_CPT)
    def _(k):
      pltpu.sync_copy(ew_v.at[k], deg_sh.at[col_v.at[k]], add=True)

    plsc.subcore_barrier()
    pltpu.sync_copy(deg_sh.at[pl.ds(s * STRIPE, STRIPE)],
                    out_hbm.at[c, pl.ds(s * STRIPE, STRIPE)])

  @functools.partial(
      pl.kernel,
      out_type=jax.ShapeDtypeStruct((NC, NPAD, HALF), jnp.float32),
      mesh=mesh,
      compiler_params=params,
      scratch_types=[
          pltpu.VMEM((CPT, CHUNK), jnp.int32),
          pltpu.VMEM((CPT, CHUNK), jnp.int32),
          pltpu.VMEM((CPT, CHUNK), jnp.float32),
          pltpu.VMEM((CHUNK, HALF), jnp.bfloat16),
          pltpu.VMEM((CHUNK, HALF), jnp.bfloat16),
          pltpu.VMEM((CHUNK, HALF), jnp.float32),
          pltpu.VMEM((CHUNK, HALF), jnp.float32),
          pltpu.VMEM_SHARED((NPAD, HALF), jnp.float32),
          pltpu.SemaphoreType.DMA,
          pltpu.SemaphoreType.DMA,
          pltpu.SemaphoreType.DMA,
          pltpu.SemaphoreType.DMA,
      ],
  )
  def agg_kernel(xs_hbm, row_hbm, col_hbm, ew_hbm, zero_hbm, out_hbm,
                 row_v, col_v, ew_v, gbuf0, gbuf1, sbuf0, sbuf1, acc_sh,
                 gsem0, gsem1, ssem0, ssem1):
    c = lax.axis_index("c")
    s = lax.axis_index("s")
    pltpu.sync_copy(row_hbm.at[s], row_v)
    pltpu.sync_copy(col_hbm.at[s], col_v)
    pltpu.sync_copy(ew_hbm.at[s], ew_v)

    base = c * N

    @pl.loop(0, CPT)
    def _(k):
      for g in range(CHUNK // LANES):
        sl = pl.ds(g * LANES, LANES)
        row_v[k, sl] = row_v[k, sl] + base

    pltpu.sync_copy(zero_hbm, acc_sh.at[pl.ds(s * STRIPE, STRIPE)])
    plsc.subcore_barrier()

    pltpu.async_copy(xs_hbm.at[row_v.at[0]], gbuf0, gsem0)
    pltpu.async_copy(xs_hbm.at[row_v.at[1]], gbuf1, gsem1)

    def process(k, gbuf, gsem, sbuf, ssem):
      pltpu.make_async_copy(xs_hbm.at[row_v.at[k]], gbuf, gsem).wait()

      @pl.when(k >= 2)
      def _():
        pltpu.make_async_copy(sbuf, acc_sh.at[col_v.at[k]], ssem).wait()

      @pl.loop(0, CHUNK // LANES)
      def _(jg):
        wv = ew_v[k, pl.ds(jg * LANES, LANES)]
        for jj in range(LANES):
          w = wv[jj]
          j = jg * LANES + jj
          for g in range(HALF // (2 * LANES)):
            u = gbuf[j, pl.ds(g * 2 * LANES, 2 * LANES)]
            a, b = plsc.unpack(u, format=plsc.PackFormat.INTERLEAVED)
            sbuf[j, pl.ds(g * 2 * LANES, LANES)] = a * w
            sbuf[j, pl.ds(g * 2 * LANES + LANES, LANES)] = b * w

      @pl.when(k + 2 < CPT)
      def _():
        pltpu.async_copy(xs_hbm.at[row_v.at[k + 2]], gbuf, gsem)

      pltpu.async_copy(sbuf, acc_sh.at[col_v.at[k]], ssem, add=True)

    @pl.loop(0, CPT, step=2)
    def _(k):
      process(k, gbuf0, gsem0, sbuf0, ssem0)
      process(k + 1, gbuf1, gsem1, sbuf1, ssem1)

    pltpu.make_async_copy(sbuf0, acc_sh.at[col_v.at[0]], ssem0).wait()
    pltpu.make_async_copy(sbuf1, acc_sh.at[col_v.at[1]], ssem1).wait()

    plsc.subcore_barrier()
    pltpu.sync_copy(acc_sh.at[pl.ds(s * STRIPE, STRIPE)],
                    out_hbm.at[c, pl.ds(s * STRIPE, STRIPE), :])

  return deg_kernel, agg_kernel


BM = 400
GRID = N // BM


def _elu(v):
  return jnp.where(v > 0, v, jnp.exp(jnp.minimum(v, 0.0)) - 1.0)


def _dis_of(degp_ref):
  deg = degp_ref[:, 0] + degp_ref[:, 1] + 2.0
  return lax.rsqrt(deg)[:, None]


def _split2(xs, ref, bref):
  for c in (0, 1):
    half = xs[:, c * HALF:(c + 1) * HALF]
    ref[c] = half
    perm = half.reshape(BM, 2, 2, LANES).transpose(0, 1, 3, 2).reshape(BM, HALF)
    bref[c] = perm.astype(jnp.bfloat16)


def _cat2(ref):
  return jnp.concatenate([ref[0], ref[1]], axis=-1)


def _mlp_body(x_ref, degp_ref, w1, b1, w2, b2, w3, b3, wg, xs2_ref, xsb_ref):
  h = _elu(jnp.dot(x_ref[:], w1[:], preferred_element_type=jnp.float32) + b1[:])
  h = _elu(jnp.dot(h, w2[:], preferred_element_type=jnp.float32) + b2[:])
  h = _elu(jnp.dot(h, w3[:], preferred_element_type=jnp.float32) + b3[:])
  xs = _dis_of(degp_ref) * jnp.dot(h, wg[:], preferred_element_type=jnp.float32)
  _split2(xs, xs2_ref, xsb_ref)


def _combine_body(accp_ref, xs2_ref, degp_ref, b_ref, wg, out2_ref, outb_ref):
  dis = _dis_of(degp_ref)
  agg = _cat2(accp_ref) + 2.0 * _cat2(xs2_ref)
  h = _elu(dis * agg + b_ref[:])
  xs = dis * jnp.dot(h, wg[:], preferred_element_type=jnp.float32)
  _split2(xs, out2_ref, outb_ref)


def _final_body(accp_ref, xs2_ref, degp_ref, b_ref, wh, bh, out_ref):
  dis = _dis_of(degp_ref)
  agg = _cat2(accp_ref) + 2.0 * _cat2(xs2_ref)
  h = _elu(dis * agg + b_ref[:])
  out_ref[:] = jnp.dot(h, wh[:], preferred_element_type=jnp.float32) + bh[:]


def _full_spec(shape):
  return pl.BlockSpec(shape, lambda i: tuple(0 for _ in shape))


_degp_spec = pl.BlockSpec((BM, 2), lambda i: (i, 0))
_half2_spec = pl.BlockSpec((2, BM, HALF), lambda i: (0, i, 0))
_xs_out_specs = (_half2_spec, _half2_spec)


def _xs_out_shapes():
  return (jax.ShapeDtypeStruct((2, N, HALF), jnp.float32),
          jax.ShapeDtypeStruct((2, N, HALF), jnp.bfloat16))


def _mlp_call(x, degp, W1, b1, W2, b2, W3, b3, Wg1):
  return pl.pallas_call(
      _mlp_body,
      grid=(GRID,),
      in_specs=[pl.BlockSpec((BM, HID), lambda i: (i, 0)), _degp_spec,
                _full_spec((HID, HID)), _full_spec((1, HID)),
                _full_spec((HID, HID)), _full_spec((1, HID)),
                _full_spec((HID, HID)), _full_spec((1, HID)),
                _full_spec((HID, HID))],
      out_specs=_xs_out_specs,
      out_shape=_xs_out_shapes(),
  )(x, degp, W1, b1, W2, b2, W3, b3, Wg1)


def _combine_call(accp, xs2, degp, b, Wg_next):
  return pl.pallas_call(
      _combine_body,
      grid=(GRID,),
      in_specs=[_half2_spec, _half2_spec, _degp_spec,
                _full_spec((1, HID)), _full_spec((HID, HID))],
      out_specs=_xs_out_specs,
      out_shape=_xs_out_shapes(),
  )(accp, xs2, degp, b, Wg_next)


def _final_call(accp, xs2, degp, b, Wh, bh):
  return pl.pallas_call(
      _final_body,
      grid=(GRID,),
      in_specs=[_half2_spec, _half2_spec, _degp_spec,
                _full_spec((1, HID)), _full_spec((HID, OUT)),
                _full_spec((1, OUT))],
      out_specs=pl.BlockSpec((BM, OUT), lambda i: (i, 0)),
      out_shape=jax.ShapeDtypeStruct((N, OUT), jnp.float32),
  )(accp, xs2, degp, b, Wh, bh)


def kernel(x, edge_index, edge_weight, W1, b1, W2, b2, W3, b3,
           Wg1, bg1, Wg2, bg2, Wg3, bg3, Wg4, bg4, Wh, bh):
  deg_kernel, agg_kernel = _sc_kernels()
  E = edge_weight.shape[0]
  pad = EPAD - E
  row = edge_index[0].astype(jnp.int32)
  col = edge_index[1].astype(jnp.int32)
  row_p = jnp.concatenate([row, jnp.zeros((pad,), jnp.int32)]).reshape(NS, CPT, CHUNK)
  col_p = jnp.concatenate([col, jnp.zeros((pad,), jnp.int32)]).reshape(NS, CPT, CHUNK)
  ew_p = jnp.concatenate(
      [edge_weight.astype(jnp.float32), jnp.zeros((pad,), jnp.float32)]
  ).reshape(NS, CPT, CHUNK)
  zero_blk = jnp.zeros((STRIPE, HALF), jnp.float32)

  b1r, b2r, b3r = b1.reshape(1, HID), b2.reshape(1, HID), b3.reshape(1, HID)
  bg = [b.reshape(1, HID) for b in (bg1, bg2, bg3, bg4)]
  bhr = bh.reshape(1, OUT)

  degp = deg_kernel(col_p, ew_p).T
  xs2, xsb = _mlp_call(x, degp, W1, b1r, W2, b2r, W3, b3r, Wg1)
  for Wn, bl in ((Wg2, bg[0]), (Wg3, bg[1]), (Wg4, bg[2])):
    accp = agg_kernel(xsb.reshape(2 * N, HALF), row_p, col_p, ew_p, zero_blk)
    xs2, xsb = _combine_call(accp, xs2, degp, bl, Wn)
  accp = agg_kernel(xsb.reshape(2 * N, HALF), row_p, col_p, ew_p, zero_blk)
  return _final_call(accp, xs2, degp, bg[3], Wh, bhr)


# speedup vs baseline: 1.2407x; 1.2407x over previous
import functools
import jax
import jax.numpy as jnp
from jax import lax
from jax.experimental import pallas as pl
from jax.experimental.pallas import tpu as pltpu
from jax.experimental.pallas import tpu_sc as plsc

N = 10000
HID = 128
OUT = 64
HALF = HID // 2
NC, NS, LANES = 2, 16, 16
STRIPE = 640
NPAD = NS * STRIPE
CHUNK = 128
CPT = 160
EPT = CHUNK * CPT
EPAD = EPT * NS
DEG_CPT = CPT // NC


@functools.lru_cache(maxsize=None)
def _sc_kernels():
  mesh = plsc.VectorSubcoreMesh(core_axis_name="c", subcore_axis_name="s",
                                num_cores=NC, num_subcores=NS)
  params = pltpu.CompilerParams(use_tc_tiling_on_sc=False)

  @functools.partial(
      pl.kernel,
      out_type=jax.ShapeDtypeStruct((NC, NPAD), jnp.float32),
      mesh=mesh,
      compiler_params=params,
      scratch_types=[
          pltpu.VMEM((DEG_CPT, CHUNK), jnp.int32),
          pltpu.VMEM((DEG_CPT, CHUNK), jnp.float32),
          pltpu.VMEM((STRIPE,), jnp.float32),
          pltpu.VMEM_SHARED((NPAD,), jnp.float32),
      ],
  )
  def deg_kernel(col_hbm, ew_hbm, out_hbm, col_v, ew_v, zeros_v, deg_sh):
    c = lax.axis_index("c")
    s = lax.axis_index("s")
    pltpu.sync_copy(col_hbm.at[s, pl.ds(c * DEG_CPT, DEG_CPT)], col_v)
    pltpu.sync_copy(ew_hbm.at[s, pl.ds(c * DEG_CPT, DEG_CPT)], ew_v)

    @pl.loop(0, STRIPE // LANES)
    def _(j):
      zeros_v[pl.ds(j * LANES, LANES)] = jnp.zeros((LANES,), jnp.float32)

    pltpu.sync_copy(zeros_v, deg_sh.at[pl.ds(s * STRIPE, STRIPE)])
    plsc.subcore_barrier()

    @pl.loop(0, DEG_CPT)
    def _(k):
      pltpu.sync_copy(ew_v.at[k], deg_sh.at[col_v.at[k]], add=True)

    plsc.subcore_barrier()
    pltpu.sync_copy(deg_sh.at[pl.ds(s * STRIPE, STRIPE)],
                    out_hbm.at[c, pl.ds(s * STRIPE, STRIPE)])

  @functools.partial(
      pl.kernel,
      out_type=jax.ShapeDtypeStruct((NC, NPAD, HALF), jnp.float32),
      mesh=mesh,
      compiler_params=params,
      scratch_types=[
          pltpu.VMEM((CPT, CHUNK), jnp.int32),
          pltpu.VMEM((CPT, CHUNK), jnp.int32),
          pltpu.VMEM((CPT, CHUNK), jnp.float32),
          pltpu.VMEM((CHUNK, HALF), jnp.float32),
          pltpu.VMEM((CHUNK, HALF), jnp.float32),
          pltpu.VMEM((CHUNK, HALF), jnp.float32),
          pltpu.VMEM_SHARED((NPAD, HALF), jnp.float32),
          pltpu.SemaphoreType.DMA,
          pltpu.SemaphoreType.DMA,
          pltpu.SemaphoreType.DMA,
      ],
  )
  def agg_kernel(xs_hbm, row_hbm, col_hbm, ew_hbm, zero_hbm, out_hbm,
                 row_v, col_v, ew_v, gbuf0, gbuf1, sbuf, acc_sh,
                 gsem0, gsem1, ssem):
    c = lax.axis_index("c")
    s = lax.axis_index("s")
    pltpu.sync_copy(row_hbm.at[s], row_v)
    pltpu.sync_copy(col_hbm.at[s], col_v)
    pltpu.sync_copy(ew_hbm.at[s], ew_v)

    base = c * N

    @pl.loop(0, CPT)
    def _(k):
      for g in range(CHUNK // LANES):
        sl = pl.ds(g * LANES, LANES)
        row_v[k, sl] = row_v[k, sl] + base

    pltpu.sync_copy(zero_hbm, acc_sh.at[pl.ds(s * STRIPE, STRIPE)])
    plsc.subcore_barrier()

    pltpu.async_copy(xs_hbm.at[row_v.at[0]], gbuf0, gsem0)
    pltpu.async_copy(xs_hbm.at[row_v.at[1]], gbuf1, gsem1)

    def process(k, gbuf, gsem):
      pltpu.make_async_copy(xs_hbm.at[row_v.at[k]], gbuf, gsem).wait()

      @pl.when(k >= 1)
      def _():
        pltpu.make_async_copy(sbuf, acc_sh.at[col_v.at[k]], ssem).wait()

      @pl.loop(0, CHUNK // LANES)
      def _(jg):
        wv = ew_v[k, pl.ds(jg * LANES, LANES)]
        for jj in range(LANES):
          w = wv[jj]
          j = jg * LANES + jj
          for g in range(HALF // LANES):
            sl = pl.ds(g * LANES, LANES)
            sbuf[j, sl] = gbuf[j, sl] * w

      @pl.when(k + 2 < CPT)
      def _():
        pltpu.async_copy(xs_hbm.at[row_v.at[k + 2]], gbuf, gsem)

      pltpu.async_copy(sbuf, acc_sh.at[col_v.at[k]], ssem, add=True)

    @pl.loop(0, CPT, step=2)
    def _(k):
      process(k, gbuf0, gsem0)
      process(k + 1, gbuf1, gsem1)

    pltpu.make_async_copy(sbuf, acc_sh.at[col_v.at[0]], ssem).wait()

    plsc.subcore_barrier()
    pltpu.sync_copy(acc_sh.at[pl.ds(s * STRIPE, STRIPE)],
                    out_hbm.at[c, pl.ds(s * STRIPE, STRIPE), :])

  return deg_kernel, agg_kernel


BM = 400
GRID = N // BM


def _elu(v):
  return jnp.where(v > 0, v, jnp.exp(jnp.minimum(v, 0.0)) - 1.0)


def _dis_of(degp_ref):
  deg = degp_ref[:, 0] + degp_ref[:, 1] + 2.0
  return lax.rsqrt(deg)[:, None]


def _split2(xs, ref):
  ref[0] = xs[:, :HALF]
  ref[1] = xs[:, HALF:]


def _cat2(ref):
  return jnp.concatenate([ref[0], ref[1]], axis=-1)


def _mlp_body(x_ref, degp_ref, w1, b1, w2, b2, w3, b3, wg, xs2_ref):
  h = _elu(jnp.dot(x_ref[:], w1[:], preferred_element_type=jnp.float32) + b1[:])
  h = _elu(jnp.dot(h, w2[:], preferred_element_type=jnp.float32) + b2[:])
  h = _elu(jnp.dot(h, w3[:], preferred_element_type=jnp.float32) + b3[:])
  xs = _dis_of(degp_ref) * jnp.dot(h, wg[:], preferred_element_type=jnp.float32)
  _split2(xs, xs2_ref)


def _combine_body(accp_ref, xs2_ref, degp_ref, b_ref, wg, out2_ref):
  dis = _dis_of(degp_ref)
  agg = _cat2(accp_ref) + 2.0 * _cat2(xs2_ref)
  h = _elu(dis * agg + b_ref[:])
  _split2(dis * jnp.dot(h, wg[:], preferred_element_type=jnp.float32), out2_ref)


def _final_body(accp_ref, xs2_ref, degp_ref, b_ref, wh, bh, out_ref):
  dis = _dis_of(degp_ref)
  agg = _cat2(accp_ref) + 2.0 * _cat2(xs2_ref)
  h = _elu(dis * agg + b_ref[:])
  out_ref[:] = jnp.dot(h, wh[:], preferred_element_type=jnp.float32) + bh[:]


def _full_spec(shape):
  return pl.BlockSpec(shape, lambda i: tuple(0 for _ in shape))


_degp_spec = pl.BlockSpec((BM, 2), lambda i: (i, 0))
_half2_spec = pl.BlockSpec((2, BM, HALF), lambda i: (0, i, 0))
_xs_out_specs = _half2_spec


def _xs_out_shapes():
  return jax.ShapeDtypeStruct((2, N, HALF), jnp.float32)


def _mlp_call(x, degp, W1, b1, W2, b2, W3, b3, Wg1):
  return pl.pallas_call(
      _mlp_body,
      grid=(GRID,),
      in_specs=[pl.BlockSpec((BM, HID), lambda i: (i, 0)), _degp_spec,
                _full_spec((HID, HID)), _full_spec((1, HID)),
                _full_spec((HID, HID)), _full_spec((1, HID)),
                _full_spec((HID, HID)), _full_spec((1, HID)),
                _full_spec((HID, HID))],
      out_specs=_xs_out_specs,
      out_shape=_xs_out_shapes(),
  )(x, degp, W1, b1, W2, b2, W3, b3, Wg1)


def _combine_call(accp, xs2, degp, b, Wg_next):
  return pl.pallas_call(
      _combine_body,
      grid=(GRID,),
      in_specs=[_half2_spec, _half2_spec, _degp_spec,
                _full_spec((1, HID)), _full_spec((HID, HID))],
      out_specs=_xs_out_specs,
      out_shape=_xs_out_shapes(),
  )(accp, xs2, degp, b, Wg_next)


def _final_call(accp, xs2, degp, b, Wh, bh):
  return pl.pallas_call(
      _final_body,
      grid=(GRID,),
      in_specs=[_half2_spec, _half2_spec, _degp_spec,
                _full_spec((1, HID)), _full_spec((HID, OUT)),
                _full_spec((1, OUT))],
      out_specs=pl.BlockSpec((BM, OUT), lambda i: (i, 0)),
      out_shape=jax.ShapeDtypeStruct((N, OUT), jnp.float32),
  )(accp, xs2, degp, b, Wh, bh)


def kernel(x, edge_index, edge_weight, W1, b1, W2, b2, W3, b3,
           Wg1, bg1, Wg2, bg2, Wg3, bg3, Wg4, bg4, Wh, bh):
  deg_kernel, agg_kernel = _sc_kernels()
  E = edge_weight.shape[0]
  pad = EPAD - E
  row = edge_index[0].astype(jnp.int32)
  col = edge_index[1].astype(jnp.int32)
  row_p = jnp.concatenate([row, jnp.zeros((pad,), jnp.int32)]).reshape(NS, CPT, CHUNK)
  col_p = jnp.concatenate([col, jnp.zeros((pad,), jnp.int32)]).reshape(NS, CPT, CHUNK)
  ew_p = jnp.concatenate(
      [edge_weight.astype(jnp.float32), jnp.zeros((pad,), jnp.float32)]
  ).reshape(NS, CPT, CHUNK)
  zero_blk = jnp.zeros((STRIPE, HALF), jnp.float32)

  b1r, b2r, b3r = b1.reshape(1, HID), b2.reshape(1, HID), b3.reshape(1, HID)
  bg = [b.reshape(1, HID) for b in (bg1, bg2, bg3, bg4)]
  bhr = bh.reshape(1, OUT)

  degp = deg_kernel(col_p, ew_p).T
  xs2 = _mlp_call(x, degp, W1, b1r, W2, b2r, W3, b3r, Wg1)
  for Wn, bl in ((Wg2, bg[0]), (Wg3, bg[1]), (Wg4, bg[2])):
    accp = agg_kernel(xs2.reshape(2 * N, HALF), row_p, col_p, ew_p, zero_blk)
    xs2 = _combine_call(accp, xs2, degp, bl, Wn)
  accp = agg_kernel(xs2.reshape(2 * N, HALF), row_p, col_p, ew_p, zero_blk)
  return _final_call(accp, xs2, degp, bg[3], Wh, bhr)


# speedup vs baseline: 1.2463x; 1.0045x over previous
import functools
import jax
import jax.numpy as jnp
from jax import lax
from jax.experimental import pallas as pl
from jax.experimental.pallas import tpu as pltpu
from jax.experimental.pallas import tpu_sc as plsc

N = 10000
HID = 128
OUT = 64
HALF = HID // 2
NC, NS, LANES = 2, 16, 16
STRIPE = 640
NPAD = NS * STRIPE
CHUNK = 128
CPT = 160
EPT = CHUNK * CPT
EPAD = EPT * NS
DEG_CPT = CPT // NC


@functools.lru_cache(maxsize=None)
def _sc_kernels():
  mesh = plsc.VectorSubcoreMesh(core_axis_name="c", subcore_axis_name="s",
                                num_cores=NC, num_subcores=NS)
  params = pltpu.CompilerParams(use_tc_tiling_on_sc=False)

  @functools.partial(
      pl.kernel,
      out_type=jax.ShapeDtypeStruct((NC, NPAD), jnp.float32),
      mesh=mesh,
      compiler_params=params,
      scratch_types=[
          pltpu.VMEM((DEG_CPT, CHUNK), jnp.int32),
          pltpu.VMEM((DEG_CPT, CHUNK), jnp.float32),
          pltpu.VMEM((STRIPE,), jnp.float32),
          pltpu.VMEM_SHARED((NPAD,), jnp.float32),
      ],
  )
  def deg_kernel(col_hbm, ew_hbm, out_hbm, col_v, ew_v, zeros_v, deg_sh):
    c = lax.axis_index("c")
    s = lax.axis_index("s")
    pltpu.sync_copy(col_hbm.at[s, pl.ds(c * DEG_CPT, DEG_CPT)], col_v)
    pltpu.sync_copy(ew_hbm.at[s, pl.ds(c * DEG_CPT, DEG_CPT)], ew_v)

    @pl.loop(0, STRIPE // LANES)
    def _(j):
      zeros_v[pl.ds(j * LANES, LANES)] = jnp.zeros((LANES,), jnp.float32)

    pltpu.sync_copy(zeros_v, deg_sh.at[pl.ds(s * STRIPE, STRIPE)])
    plsc.subcore_barrier()

    @pl.loop(0, DEG_CPT)
    def _(k):
      pltpu.sync_copy(ew_v.at[k], deg_sh.at[col_v.at[k]], add=True)

    plsc.subcore_barrier()
    pltpu.sync_copy(deg_sh.at[pl.ds(s * STRIPE, STRIPE)],
                    out_hbm.at[c, pl.ds(s * STRIPE, STRIPE)])

  @functools.partial(
      pl.kernel,
      out_type=jax.ShapeDtypeStruct((NC, NPAD, HALF), jnp.float32),
      mesh=mesh,
      compiler_params=params,
      scratch_types=[
          pltpu.VMEM((CPT, CHUNK), jnp.int32),
          pltpu.VMEM((CPT, CHUNK), jnp.int32),
          pltpu.VMEM((CPT, CHUNK), jnp.float32),
          pltpu.VMEM((CHUNK, HALF), jnp.float32),
          pltpu.VMEM((CHUNK, HALF), jnp.float32),
          pltpu.VMEM((CHUNK, HALF), jnp.float32),
          pltpu.VMEM_SHARED((NPAD, HALF), jnp.float32),
          pltpu.SemaphoreType.DMA,
          pltpu.SemaphoreType.DMA,
          pltpu.SemaphoreType.DMA,
      ],
  )
  def agg_kernel(xs_hbm, row_hbm, col_hbm, ew_hbm, zero_hbm, out_hbm,
                 row_v, col_v, ew_v, gbuf0, gbuf1, sbuf, acc_sh,
                 gsem0, gsem1, ssem):
    c = lax.axis_index("c")
    s = lax.axis_index("s")
    pltpu.sync_copy(row_hbm.at[s], row_v)
    pltpu.sync_copy(col_hbm.at[s], col_v)
    pltpu.sync_copy(ew_hbm.at[s], ew_v)

    base = c * N

    @pl.loop(0, CPT)
    def _(k):
      for g in range(CHUNK // LANES):
        sl = pl.ds(g * LANES, LANES)
        row_v[k, sl] = row_v[k, sl] + base

    pltpu.sync_copy(zero_hbm, acc_sh.at[pl.ds(s * STRIPE, STRIPE)])
    plsc.subcore_barrier()

    pltpu.async_copy(xs_hbm.at[row_v.at[0]], gbuf0, gsem0)
    pltpu.async_copy(xs_hbm.at[row_v.at[1]], gbuf1, gsem1)

    def process(k, gbuf, gsem):
      pltpu.make_async_copy(xs_hbm.at[row_v.at[k]], gbuf, gsem).wait()

      @pl.when(k >= 1)
      def _():
        pltpu.make_async_copy(sbuf, acc_sh.at[col_v.at[k]], ssem).wait()

      @plsc.parallel_loop(0, CHUNK // LANES, unroll=2)
      def _(jg):
        wv = ew_v[k, pl.ds(jg * LANES, LANES)]
        for jj in range(LANES):
          w = wv[jj]
          j = jg * LANES + jj
          for g in range(HALF // LANES):
            sl = pl.ds(g * LANES, LANES)
            sbuf[j, sl] = gbuf[j, sl] * w

      @pl.when(k + 2 < CPT)
      def _():
        pltpu.async_copy(xs_hbm.at[row_v.at[k + 2]], gbuf, gsem)

      pltpu.async_copy(sbuf, acc_sh.at[col_v.at[k]], ssem, add=True)

    @pl.loop(0, CPT, step=2)
    def _(k):
      process(k, gbuf0, gsem0)
      process(k + 1, gbuf1, gsem1)

    pltpu.make_async_copy(sbuf, acc_sh.at[col_v.at[0]], ssem).wait()

    plsc.subcore_barrier()
    pltpu.sync_copy(acc_sh.at[pl.ds(s * STRIPE, STRIPE)],
                    out_hbm.at[c, pl.ds(s * STRIPE, STRIPE), :])

  return deg_kernel, agg_kernel


BM = 400
GRID = N // BM


def _elu(v):
  return jnp.where(v > 0, v, jnp.exp(jnp.minimum(v, 0.0)) - 1.0)


def _dis_of(degp_ref):
  deg = degp_ref[:, 0] + degp_ref[:, 1] + 2.0
  return lax.rsqrt(deg)[:, None]


def _split2(xs, ref):
  ref[0] = xs[:, :HALF]
  ref[1] = xs[:, HALF:]


def _cat2(ref):
  return jnp.concatenate([ref[0], ref[1]], axis=-1)


def _mlp_body(x_ref, degp_ref, w1, b1, w2, b2, w3, b3, wg, xs2_ref):
  h = _elu(jnp.dot(x_ref[:], w1[:], preferred_element_type=jnp.float32) + b1[:])
  h = _elu(jnp.dot(h, w2[:], preferred_element_type=jnp.float32) + b2[:])
  h = _elu(jnp.dot(h, w3[:], preferred_element_type=jnp.float32) + b3[:])
  xs = _dis_of(degp_ref) * jnp.dot(h, wg[:], preferred_element_type=jnp.float32)
  _split2(xs, xs2_ref)


def _combine_body(accp_ref, xs2_ref, degp_ref, b_ref, wg, out2_ref):
  dis = _dis_of(degp_ref)
  agg = _cat2(accp_ref) + 2.0 * _cat2(xs2_ref)
  h = _elu(dis * agg + b_ref[:])
  _split2(dis * jnp.dot(h, wg[:], preferred_element_type=jnp.float32), out2_ref)


def _final_body(accp_ref, xs2_ref, degp_ref, b_ref, wh, bh, out_ref):
  dis = _dis_of(degp_ref)
  agg = _cat2(accp_ref) + 2.0 * _cat2(xs2_ref)
  h = _elu(dis * agg + b_ref[:])
  out_ref[:] = jnp.dot(h, wh[:], preferred_element_type=jnp.float32) + bh[:]


def _full_spec(shape):
  return pl.BlockSpec(shape, lambda i: tuple(0 for _ in shape))


_degp_spec = pl.BlockSpec((BM, 2), lambda i: (i, 0))
_half2_spec = pl.BlockSpec((2, BM, HALF), lambda i: (0, i, 0))
_xs_out_specs = _half2_spec


def _xs_out_shapes():
  return jax.ShapeDtypeStruct((2, N, HALF), jnp.float32)


def _mlp_call(x, degp, W1, b1, W2, b2, W3, b3, Wg1):
  return pl.pallas_call(
      _mlp_body,
      grid=(GRID,),
      in_specs=[pl.BlockSpec((BM, HID), lambda i: (i, 0)), _degp_spec,
                _full_spec((HID, HID)), _full_spec((1, HID)),
                _full_spec((HID, HID)), _full_spec((1, HID)),
                _full_spec((HID, HID)), _full_spec((1, HID)),
                _full_spec((HID, HID))],
      out_specs=_xs_out_specs,
      out_shape=_xs_out_shapes(),
  )(x, degp, W1, b1, W2, b2, W3, b3, Wg1)


def _combine_call(accp, xs2, degp, b, Wg_next):
  return pl.pallas_call(
      _combine_body,
      grid=(GRID,),
      in_specs=[_half2_spec, _half2_spec, _degp_spec,
                _full_spec((1, HID)), _full_spec((HID, HID))],
      out_specs=_xs_out_specs,
      out_shape=_xs_out_shapes(),
  )(accp, xs2, degp, b, Wg_next)


def _final_call(accp, xs2, degp, b, Wh, bh):
  return pl.pallas_call(
      _final_body,
      grid=(GRID,),
      in_specs=[_half2_spec, _half2_spec, _degp_spec,
                _full_spec((1, HID)), _full_spec((HID, OUT)),
                _full_spec((1, OUT))],
      out_specs=pl.BlockSpec((BM, OUT), lambda i: (i, 0)),
      out_shape=jax.ShapeDtypeStruct((N, OUT), jnp.float32),
  )(accp, xs2, degp, b, Wh, bh)


def kernel(x, edge_index, edge_weight, W1, b1, W2, b2, W3, b3,
           Wg1, bg1, Wg2, bg2, Wg3, bg3, Wg4, bg4, Wh, bh):
  deg_kernel, agg_kernel = _sc_kernels()
  E = edge_weight.shape[0]
  pad = EPAD - E
  row = edge_index[0].astype(jnp.int32)
  col = edge_index[1].astype(jnp.int32)
  row_p = jnp.concatenate([row, jnp.zeros((pad,), jnp.int32)]).reshape(NS, CPT, CHUNK)
  col_p = jnp.concatenate([col, jnp.zeros((pad,), jnp.int32)]).reshape(NS, CPT, CHUNK)
  ew_p = jnp.concatenate(
      [edge_weight.astype(jnp.float32), jnp.zeros((pad,), jnp.float32)]
  ).reshape(NS, CPT, CHUNK)
  zero_blk = jnp.zeros((STRIPE, HALF), jnp.float32)

  b1r, b2r, b3r = b1.reshape(1, HID), b2.reshape(1, HID), b3.reshape(1, HID)
  bg = [b.reshape(1, HID) for b in (bg1, bg2, bg3, bg4)]
  bhr = bh.reshape(1, OUT)

  degp = deg_kernel(col_p, ew_p).T
  xs2 = _mlp_call(x, degp, W1, b1r, W2, b2r, W3, b3r, Wg1)
  for Wn, bl in ((Wg2, bg[0]), (Wg3, bg[1]), (Wg4, bg[2])):
    accp = agg_kernel(xs2.reshape(2 * N, HALF), row_p, col_p, ew_p, zero_blk)
    xs2 = _combine_call(accp, xs2, degp, bl, Wn)
  accp = agg_kernel(xs2.reshape(2 * N, HALF), row_p, col_p, ew_p, zero_blk)
  return _final_call(accp, xs2, degp, bg[3], Wh, bhr)


# speedup vs baseline: 1.9172x; 1.5384x over previous
import functools
import jax
import jax.numpy as jnp
from jax import lax
from jax.experimental import pallas as pl
from jax.experimental.pallas import tpu as pltpu
from jax.experimental.pallas import tpu_sc as plsc

N = 10000
HID = 128
OUT = 64
HALF = HID // 2
NC, NS, LANES = 2, 16, 16
STRIPE = 640
NPAD = NS * STRIPE
CHUNK = 128
CPT = 160
EPT = CHUNK * CPT
EPAD = EPT * NS
DEG_CPT = CPT // NC
QB = 20
NBLK = CPT // QB
TROWS = N // NS


@functools.lru_cache(maxsize=None)
def _sc_kernels():
  mesh = plsc.VectorSubcoreMesh(core_axis_name="c", subcore_axis_name="s",
                                num_cores=NC, num_subcores=NS)
  params = pltpu.CompilerParams(use_tc_tiling_on_sc=False)

  @functools.partial(
      pl.kernel,
      out_type=jax.ShapeDtypeStruct((NC, NPAD), jnp.float32),
      mesh=mesh,
      compiler_params=params,
      scratch_types=[
          pltpu.VMEM((DEG_CPT, CHUNK), jnp.int32),
          pltpu.VMEM((DEG_CPT, CHUNK), jnp.float32),
          pltpu.VMEM((STRIPE,), jnp.float32),
          pltpu.VMEM_SHARED((NPAD,), jnp.float32),
      ],
  )
  def deg_kernel(col_hbm, ew_hbm, out_hbm, col_v, ew_v, zeros_v, deg_sh):
    c = lax.axis_index("c")
    s = lax.axis_index("s")
    pltpu.sync_copy(col_hbm.at[s, pl.ds(c * DEG_CPT, DEG_CPT)], col_v)
    pltpu.sync_copy(ew_hbm.at[s, pl.ds(c * DEG_CPT, DEG_CPT)], ew_v)

    @pl.loop(0, STRIPE // LANES)
    def _(j):
      zeros_v[pl.ds(j * LANES, LANES)] = jnp.zeros((LANES,), jnp.float32)

    pltpu.sync_copy(zeros_v, deg_sh.at[pl.ds(s * STRIPE, STRIPE)])
    plsc.subcore_barrier()

    @pl.loop(0, DEG_CPT)
    def _(k):
      pltpu.sync_copy(ew_v.at[k], deg_sh.at[col_v.at[k]], add=True)

    plsc.subcore_barrier()
    pltpu.sync_copy(deg_sh.at[pl.ds(s * STRIPE, STRIPE)],
                    out_hbm.at[c, pl.ds(s * STRIPE, STRIPE)])

  @functools.partial(
      pl.kernel,
      out_type=jax.ShapeDtypeStruct((NC, NPAD, HALF), jnp.float32),
      mesh=mesh,
      compiler_params=params,
      scratch_types=[
          pltpu.VMEM((QB, CHUNK), jnp.int32),
          pltpu.VMEM((QB, CHUNK), jnp.int32),
          pltpu.VMEM((QB, CHUNK), jnp.int32),
          pltpu.VMEM((QB, CHUNK), jnp.int32),
          pltpu.VMEM((QB, CHUNK), jnp.float32),
          pltpu.VMEM((QB, CHUNK), jnp.float32),
          pltpu.VMEM((CHUNK, HALF), jnp.float32),
          pltpu.VMEM((CHUNK, HALF), jnp.float32),
          pltpu.VMEM((CHUNK, HALF), jnp.float32),
          pltpu.VMEM_SHARED((N, HALF), jnp.float32),
          pltpu.VMEM_SHARED((NPAD, HALF), jnp.float32),
          pltpu.SemaphoreType.DMA,
          pltpu.SemaphoreType.DMA,
          pltpu.SemaphoreType.DMA,
          pltpu.SemaphoreType.DMA,
          pltpu.SemaphoreType.DMA,
      ],
  )
  def agg_kernel(xs_hbm, row_hbm, col_hbm, ew_hbm, zero_hbm, out_hbm,
                 rowq0, rowq1, colq0, colq1, ewq0, ewq1, gbuf0, gbuf1, sbuf,
                 table_sh, acc_sh, gsem0, gsem1, ssem, msem0, msem1):
    c = lax.axis_index("c")
    s = lax.axis_index("s")
    pltpu.sync_copy(xs_hbm.at[pl.ds(c * N + s * TROWS, TROWS)],
                    table_sh.at[pl.ds(s * TROWS, TROWS)])
    pltpu.sync_copy(zero_hbm, acc_sh.at[pl.ds(s * STRIPE, STRIPE)])
    pltpu.sync_copy(row_hbm.at[s, pl.ds(0, QB)], rowq0)
    pltpu.sync_copy(col_hbm.at[s, pl.ds(0, QB)], colq0)
    pltpu.sync_copy(ew_hbm.at[s, pl.ds(0, QB)], ewq0)
    plsc.subcore_barrier()

    bufs = ((rowq0, colq0, ewq0, msem0), (rowq1, colq1, ewq1, msem1))

    def process(k, gbuf, gsem, rq, cq, eq, first):
      pltpu.make_async_copy(table_sh.at[rq.at[k]], gbuf, gsem).wait()

      def wait_scat():
        pltpu.make_async_copy(sbuf, acc_sh.at[cq.at[k]], ssem).wait()
      if first:
        @pl.when(k >= 1)
        def _():
          wait_scat()
      else:
        wait_scat()

      @plsc.parallel_loop(0, CHUNK // LANES)
      def _(jg):
        wv = eq[k, pl.ds(jg * LANES, LANES)]
        for jj in range(LANES):
          w = wv[jj]
          j = jg * LANES + jj
          for g in range(HALF // LANES):
            sl = pl.ds(g * LANES, LANES)
            sbuf[j, sl] = gbuf[j, sl] * w

      @pl.when(k + 2 < QB)
      def _():
        pltpu.async_copy(table_sh.at[rq.at[k + 2]], gbuf, gsem)

      pltpu.async_copy(sbuf, acc_sh.at[cq.at[k]], ssem, add=True)

    for b in range(NBLK):
      rq, cq, eq, _ = bufs[b % 2]
      nrq, ncq, neq, nmsem = bufs[(b + 1) % 2]
      if b + 1 < NBLK:
        off = (b + 1) * QB
        pltpu.async_copy(row_hbm.at[s, pl.ds(off, QB)], nrq, nmsem)
        pltpu.async_copy(col_hbm.at[s, pl.ds(off, QB)], ncq, nmsem)
        pltpu.async_copy(ew_hbm.at[s, pl.ds(off, QB)], neq, nmsem)

      pltpu.async_copy(table_sh.at[rq.at[0]], gbuf0, gsem0)
      pltpu.async_copy(table_sh.at[rq.at[1]], gbuf1, gsem1)

      @pl.loop(0, QB, step=2)
      def _(k):
        process(k, gbuf0, gsem0, rq, cq, eq, b == 0)
        process(k + 1, gbuf1, gsem1, rq, cq, eq, b == 0)

      if b + 1 < NBLK:
        off = (b + 1) * QB
        pltpu.make_async_copy(row_hbm.at[s, pl.ds(off, QB)], nrq, nmsem).wait()
        pltpu.make_async_copy(col_hbm.at[s, pl.ds(off, QB)], ncq, nmsem).wait()
        pltpu.make_async_copy(ew_hbm.at[s, pl.ds(off, QB)], neq, nmsem).wait()

    pltpu.make_async_copy(sbuf, acc_sh.at[colq1.at[0]], ssem).wait()

    plsc.subcore_barrier()
    pltpu.sync_copy(acc_sh.at[pl.ds(s * STRIPE, STRIPE)],
                    out_hbm.at[c, pl.ds(s * STRIPE, STRIPE), :])

  return deg_kernel, agg_kernel


BM = 400
GRID = N // BM


def _elu(v):
  return jnp.where(v > 0, v, jnp.exp(jnp.minimum(v, 0.0)) - 1.0)


def _dis_of(degp_ref):
  deg = degp_ref[:, 0] + degp_ref[:, 1] + 2.0
  return lax.rsqrt(deg)[:, None]


def _split2(xs, ref):
  ref[0] = xs[:, :HALF]
  ref[1] = xs[:, HALF:]


def _cat2(ref):
  return jnp.concatenate([ref[0], ref[1]], axis=-1)


def _mlp_body(x_ref, degp_ref, w1, b1, w2, b2, w3, b3, wg, xs2_ref):
  h = _elu(jnp.dot(x_ref[:], w1[:], preferred_element_type=jnp.float32) + b1[:])
  h = _elu(jnp.dot(h, w2[:], preferred_element_type=jnp.float32) + b2[:])
  h = _elu(jnp.dot(h, w3[:], preferred_element_type=jnp.float32) + b3[:])
  xs = _dis_of(degp_ref) * jnp.dot(h, wg[:], preferred_element_type=jnp.float32)
  _split2(xs, xs2_ref)


def _combine_body(accp_ref, xs2_ref, degp_ref, b_ref, wg, out2_ref):
  dis = _dis_of(degp_ref)
  agg = _cat2(accp_ref) + 2.0 * _cat2(xs2_ref)
  h = _elu(dis * agg + b_ref[:])
  _split2(dis * jnp.dot(h, wg[:], preferred_element_type=jnp.float32), out2_ref)


def _final_body(accp_ref, xs2_ref, degp_ref, b_ref, wh, bh, out_ref):
  dis = _dis_of(degp_ref)
  agg = _cat2(accp_ref) + 2.0 * _cat2(xs2_ref)
  h = _elu(dis * agg + b_ref[:])
  out_ref[:] = jnp.dot(h, wh[:], preferred_element_type=jnp.float32) + bh[:]


def _full_spec(shape):
  return pl.BlockSpec(shape, lambda i: tuple(0 for _ in shape))


_degp_spec = pl.BlockSpec((BM, 2), lambda i: (i, 0))
_half2_spec = pl.BlockSpec((2, BM, HALF), lambda i: (0, i, 0))
_xs_out_specs = _half2_spec


def _xs_out_shapes():
  return jax.ShapeDtypeStruct((2, N, HALF), jnp.float32)


def _mlp_call(x, degp, W1, b1, W2, b2, W3, b3, Wg1):
  return pl.pallas_call(
      _mlp_body,
      grid=(GRID,),
      in_specs=[pl.BlockSpec((BM, HID), lambda i: (i, 0)), _degp_spec,
                _full_spec((HID, HID)), _full_spec((1, HID)),
                _full_spec((HID, HID)), _full_spec((1, HID)),
                _full_spec((HID, HID)), _full_spec((1, HID)),
                _full_spec((HID, HID))],
      out_specs=_xs_out_specs,
      out_shape=_xs_out_shapes(),
  )(x, degp, W1, b1, W2, b2, W3, b3, Wg1)


def _combine_call(accp, xs2, degp, b, Wg_next):
  return pl.pallas_call(
      _combine_body,
      grid=(GRID,),
      in_specs=[_half2_spec, _half2_spec, _degp_spec,
                _full_spec((1, HID)), _full_spec((HID, HID))],
      out_specs=_xs_out_specs,
      out_shape=_xs_out_shapes(),
  )(accp, xs2, degp, b, Wg_next)


def _final_call(accp, xs2, degp, b, Wh, bh):
  return pl.pallas_call(
      _final_body,
      grid=(GRID,),
      in_specs=[_half2_spec, _half2_spec, _degp_spec,
                _full_spec((1, HID)), _full_spec((HID, OUT)),
                _full_spec((1, OUT))],
      out_specs=pl.BlockSpec((BM, OUT), lambda i: (i, 0)),
      out_shape=jax.ShapeDtypeStruct((N, OUT), jnp.float32),
  )(accp, xs2, degp, b, Wh, bh)


def kernel(x, edge_index, edge_weight, W1, b1, W2, b2, W3, b3,
           Wg1, bg1, Wg2, bg2, Wg3, bg3, Wg4, bg4, Wh, bh):
  deg_kernel, agg_kernel = _sc_kernels()
  E = edge_weight.shape[0]
  pad = EPAD - E
  row = edge_index[0].astype(jnp.int32)
  col = edge_index[1].astype(jnp.int32)
  row_p = jnp.concatenate([row, jnp.zeros((pad,), jnp.int32)]).reshape(NS, CPT, CHUNK)
  col_p = jnp.concatenate([col, jnp.zeros((pad,), jnp.int32)]).reshape(NS, CPT, CHUNK)
  ew_p = jnp.concatenate(
      [edge_weight.astype(jnp.float32), jnp.zeros((pad,), jnp.float32)]
  ).reshape(NS, CPT, CHUNK)
  zero_blk = jnp.zeros((STRIPE, HALF), jnp.float32)

  b1r, b2r, b3r = b1.reshape(1, HID), b2.reshape(1, HID), b3.reshape(1, HID)
  bg = [b.reshape(1, HID) for b in (bg1, bg2, bg3, bg4)]
  bhr = bh.reshape(1, OUT)

  degp = deg_kernel(col_p, ew_p).T
  xs2 = _mlp_call(x, degp, W1, b1r, W2, b2r, W3, b3r, Wg1)
  for Wn, bl in ((Wg2, bg[0]), (Wg3, bg[1]), (Wg4, bg[2])):
    accp = agg_kernel(xs2.reshape(2 * N, HALF), row_p, col_p, ew_p, zero_blk)
    xs2 = _combine_call(accp, xs2, degp, bl, Wn)
  accp = agg_kernel(xs2.reshape(2 * N, HALF), row_p, col_p, ew_p, zero_blk)
  return _final_call(accp, xs2, degp, bg[3], Wh, bhr)


# speedup vs baseline: 2.0232x; 1.0553x over previous
import functools
import jax
import jax.numpy as jnp
from jax import lax
from jax.experimental import pallas as pl
from jax.experimental.pallas import tpu as pltpu
from jax.experimental.pallas import tpu_sc as plsc

N = 10000
HID = 128
OUT = 64
HALF = HID // 2
NC, NS, LANES = 2, 16, 16
STRIPE = 640
NPAD = NS * STRIPE
CHUNK = 128
CPT = 160
EPT = CHUNK * CPT
EPAD = EPT * NS
DEG_CPT = CPT // NC
QB = 20
NBLK = CPT // QB
TROWS = N // NS


@functools.lru_cache(maxsize=None)
def _sc_kernels():
  mesh = plsc.VectorSubcoreMesh(core_axis_name="c", subcore_axis_name="s",
                                num_cores=NC, num_subcores=NS)
  params = pltpu.CompilerParams(use_tc_tiling_on_sc=False)

  @functools.partial(
      pl.kernel,
      out_type=jax.ShapeDtypeStruct((NC, NPAD), jnp.float32),
      mesh=mesh,
      compiler_params=params,
      scratch_types=[
          pltpu.VMEM((DEG_CPT, CHUNK), jnp.int32),
          pltpu.VMEM((DEG_CPT, CHUNK), jnp.float32),
          pltpu.VMEM((STRIPE,), jnp.float32),
          pltpu.VMEM_SHARED((NPAD,), jnp.float32),
      ],
  )
  def deg_kernel(col_hbm, ew_hbm, out_hbm, col_v, ew_v, zeros_v, deg_sh):
    c = lax.axis_index("c")
    s = lax.axis_index("s")
    pltpu.sync_copy(col_hbm.at[s, pl.ds(c * DEG_CPT, DEG_CPT)], col_v)
    pltpu.sync_copy(ew_hbm.at[s, pl.ds(c * DEG_CPT, DEG_CPT)], ew_v)

    @pl.loop(0, STRIPE // LANES)
    def _(j):
      zeros_v[pl.ds(j * LANES, LANES)] = jnp.zeros((LANES,), jnp.float32)

    pltpu.sync_copy(zeros_v, deg_sh.at[pl.ds(s * STRIPE, STRIPE)])
    plsc.subcore_barrier()

    @pl.loop(0, DEG_CPT)
    def _(k):
      pltpu.sync_copy(ew_v.at[k], deg_sh.at[col_v.at[k]], add=True)

    plsc.subcore_barrier()
    pltpu.sync_copy(deg_sh.at[pl.ds(s * STRIPE, STRIPE)],
                    out_hbm.at[c, pl.ds(s * STRIPE, STRIPE)])

  @functools.partial(
      pl.kernel,
      out_type=jax.ShapeDtypeStruct((NC, NPAD, HALF), jnp.float32),
      mesh=mesh,
      compiler_params=params,
      scratch_types=[
          pltpu.VMEM((QB, CHUNK), jnp.int32),
          pltpu.VMEM((QB, CHUNK), jnp.int32),
          pltpu.VMEM((QB, CHUNK), jnp.int32),
          pltpu.VMEM((QB, CHUNK), jnp.int32),
          pltpu.VMEM((QB, CHUNK), jnp.float32),
          pltpu.VMEM((QB, CHUNK), jnp.float32),
          pltpu.VMEM((CHUNK, HALF), jnp.float32),
          pltpu.VMEM((CHUNK, HALF), jnp.float32),
          pltpu.VMEM((CHUNK, HALF), jnp.float32),
          pltpu.VMEM_SHARED((N, HALF), jnp.float32),
          pltpu.VMEM_SHARED((NPAD, HALF), jnp.float32),
          pltpu.SemaphoreType.DMA,
          pltpu.SemaphoreType.DMA,
          pltpu.SemaphoreType.DMA,
          pltpu.SemaphoreType.DMA,
          pltpu.SemaphoreType.DMA,
      ],
  )
  def agg_kernel(xs_hbm, row_hbm, col_hbm, ew_hbm, zero_hbm, out_hbm,
                 rowq0, rowq1, colq0, colq1, ewq0, ewq1, gbuf0, gbuf1, sbuf,
                 table_sh, acc_sh, gsem0, gsem1, ssem, msem0, msem1):
    c = lax.axis_index("c")
    s = lax.axis_index("s")
    pltpu.sync_copy(xs_hbm.at[pl.ds(c * N + s * TROWS, TROWS)],
                    table_sh.at[pl.ds(s * TROWS, TROWS)])
    pltpu.sync_copy(zero_hbm, acc_sh.at[pl.ds(s * STRIPE, STRIPE)])
    pltpu.sync_copy(row_hbm.at[s, pl.ds(0, QB)], rowq0)
    pltpu.sync_copy(col_hbm.at[s, pl.ds(0, QB)], colq0)
    pltpu.sync_copy(ew_hbm.at[s, pl.ds(0, QB)], ewq0)
    plsc.subcore_barrier()

    bufs = ((rowq0, colq0, ewq0, msem0), (rowq1, colq1, ewq1, msem1))

    def process(k, gbuf, gsem, rq, cq, eq, first):
      pltpu.make_async_copy(table_sh.at[rq.at[k]], gbuf, gsem).wait()

      def wait_scat():
        pltpu.make_async_copy(sbuf, acc_sh.at[cq.at[k]], ssem).wait()
      if first:
        @pl.when(k >= 1)
        def _():
          wait_scat()
      else:
        wait_scat()

      @plsc.parallel_loop(0, CHUNK // LANES)
      def _(jg):
        wv = eq[k, pl.ds(jg * LANES, LANES)]
        for jj in range(LANES):
          w = wv[jj]
          j = jg * LANES + jj
          for g in range(HALF // LANES):
            sl = pl.ds(g * LANES, LANES)
            sbuf[j, sl] = gbuf[j, sl] * w

      @pl.when(k + 2 < QB)
      def _():
        pltpu.async_copy(table_sh.at[rq.at[k + 2]], gbuf, gsem)

      pltpu.async_copy(sbuf, acc_sh.at[cq.at[k]], ssem, add=True)

    for b in range(NBLK):
      rq, cq, eq, _ = bufs[b % 2]
      nrq, ncq, neq, nmsem = bufs[(b + 1) % 2]
      if b + 1 < NBLK:
        off = (b + 1) * QB
        pltpu.async_copy(row_hbm.at[s, pl.ds(off, QB)], nrq, nmsem)
        pltpu.async_copy(col_hbm.at[s, pl.ds(off, QB)], ncq, nmsem)
        pltpu.async_copy(ew_hbm.at[s, pl.ds(off, QB)], neq, nmsem)

      pltpu.async_copy(table_sh.at[rq.at[0]], gbuf0, gsem0)
      pltpu.async_copy(table_sh.at[rq.at[1]], gbuf1, gsem1)

      @pl.loop(0, QB, step=2)
      def _(k):
        process(k, gbuf0, gsem0, rq, cq, eq, b == 0)
        process(k + 1, gbuf1, gsem1, rq, cq, eq, b == 0)

      if b + 1 < NBLK:
        off = (b + 1) * QB
        pltpu.make_async_copy(row_hbm.at[s, pl.ds(off, QB)], nrq, nmsem).wait()
        pltpu.make_async_copy(col_hbm.at[s, pl.ds(off, QB)], ncq, nmsem).wait()
        pltpu.make_async_copy(ew_hbm.at[s, pl.ds(off, QB)], neq, nmsem).wait()

    pltpu.make_async_copy(sbuf, acc_sh.at[colq1.at[0]], ssem).wait()

    plsc.subcore_barrier()
    pltpu.sync_copy(acc_sh.at[pl.ds(s * STRIPE, STRIPE)],
                    out_hbm.at[c, pl.ds(s * STRIPE, STRIPE), :])

  return deg_kernel, agg_kernel


BM = 2000
GRID = N // BM


def _elu(v):
  return jnp.where(v > 0, v, jnp.exp(jnp.minimum(v, 0.0)) - 1.0)


def _dis_of(degp_ref):
  deg = degp_ref[:, 0] + degp_ref[:, 1] + 2.0
  return lax.rsqrt(deg)[:, None]


def _split2(xs, ref):
  ref[0] = xs[:, :HALF]
  ref[1] = xs[:, HALF:]


def _cat2(ref):
  return jnp.concatenate([ref[0], ref[1]], axis=-1)


def _mlp_body(x_ref, degp_ref, w1, b1, w2, b2, w3, b3, wg, xs2_ref):
  h = _elu(jnp.dot(x_ref[:], w1[:], preferred_element_type=jnp.float32) + b1[:])
  h = _elu(jnp.dot(h, w2[:], preferred_element_type=jnp.float32) + b2[:])
  h = _elu(jnp.dot(h, w3[:], preferred_element_type=jnp.float32) + b3[:])
  xs = _dis_of(degp_ref) * jnp.dot(h, wg[:], preferred_element_type=jnp.float32)
  _split2(xs, xs2_ref)


def _combine_body(accp_ref, xs2_ref, degp_ref, b_ref, wg, out2_ref):
  dis = _dis_of(degp_ref)
  agg = _cat2(accp_ref) + 2.0 * _cat2(xs2_ref)
  h = _elu(dis * agg + b_ref[:])
  _split2(dis * jnp.dot(h, wg[:], preferred_element_type=jnp.float32), out2_ref)


def _final_body(accp_ref, xs2_ref, degp_ref, b_ref, wh, bh, out_ref):
  dis = _dis_of(degp_ref)
  agg = _cat2(accp_ref) + 2.0 * _cat2(xs2_ref)
  h = _elu(dis * agg + b_ref[:])
  out_ref[:] = jnp.dot(h, wh[:], preferred_element_type=jnp.float32) + bh[:]


def _full_spec(shape):
  return pl.BlockSpec(shape, lambda i: tuple(0 for _ in shape))


_degp_spec = pl.BlockSpec((BM, 2), lambda i: (i, 0))
_half2_spec = pl.BlockSpec((2, BM, HALF), lambda i: (0, i, 0))
_xs_out_specs = _half2_spec


def _xs_out_shapes():
  return jax.ShapeDtypeStruct((2, N, HALF), jnp.float32)


def _mlp_call(x, degp, W1, b1, W2, b2, W3, b3, Wg1):
  return pl.pallas_call(
      _mlp_body,
      grid=(GRID,),
      in_specs=[pl.BlockSpec((BM, HID), lambda i: (i, 0)), _degp_spec,
                _full_spec((HID, HID)), _full_spec((1, HID)),
                _full_spec((HID, HID)), _full_spec((1, HID)),
                _full_spec((HID, HID)), _full_spec((1, HID)),
                _full_spec((HID, HID))],
      out_specs=_xs_out_specs,
      out_shape=_xs_out_shapes(),
  )(x, degp, W1, b1, W2, b2, W3, b3, Wg1)


def _combine_call(accp, xs2, degp, b, Wg_next):
  return pl.pallas_call(
      _combine_body,
      grid=(GRID,),
      in_specs=[_half2_spec, _half2_spec, _degp_spec,
                _full_spec((1, HID)), _full_spec((HID, HID))],
      out_specs=_xs_out_specs,
      out_shape=_xs_out_shapes(),
  )(accp, xs2, degp, b, Wg_next)


def _final_call(accp, xs2, degp, b, Wh, bh):
  return pl.pallas_call(
      _final_body,
      grid=(GRID,),
      in_specs=[_half2_spec, _half2_spec, _degp_spec,
                _full_spec((1, HID)), _full_spec((HID, OUT)),
                _full_spec((1, OUT))],
      out_specs=pl.BlockSpec((BM, OUT), lambda i: (i, 0)),
      out_shape=jax.ShapeDtypeStruct((N, OUT), jnp.float32),
  )(accp, xs2, degp, b, Wh, bh)


def kernel(x, edge_index, edge_weight, W1, b1, W2, b2, W3, b3,
           Wg1, bg1, Wg2, bg2, Wg3, bg3, Wg4, bg4, Wh, bh):
  deg_kernel, agg_kernel = _sc_kernels()
  E = edge_weight.shape[0]
  pad = EPAD - E
  row = edge_index[0].astype(jnp.int32)
  col = edge_index[1].astype(jnp.int32)
  row_p = jnp.concatenate([row, jnp.zeros((pad,), jnp.int32)]).reshape(NS, CPT, CHUNK)
  col_p = jnp.concatenate([col, jnp.zeros((pad,), jnp.int32)]).reshape(NS, CPT, CHUNK)
  ew_p = jnp.concatenate(
      [edge_weight.astype(jnp.float32), jnp.zeros((pad,), jnp.float32)]
  ).reshape(NS, CPT, CHUNK)
  zero_blk = jnp.zeros((STRIPE, HALF), jnp.float32)

  b1r, b2r, b3r = b1.reshape(1, HID), b2.reshape(1, HID), b3.reshape(1, HID)
  bg = [b.reshape(1, HID) for b in (bg1, bg2, bg3, bg4)]
  bhr = bh.reshape(1, OUT)

  degp = deg_kernel(col_p, ew_p).T
  xs2 = _mlp_call(x, degp, W1, b1r, W2, b2r, W3, b3r, Wg1)
  for Wn, bl in ((Wg2, bg[0]), (Wg3, bg[1]), (Wg4, bg[2])):
    accp = agg_kernel(xs2.reshape(2 * N, HALF), row_p, col_p, ew_p, zero_blk)
    xs2 = _combine_call(accp, xs2, degp, bl, Wn)
  accp = agg_kernel(xs2.reshape(2 * N, HALF), row_p, col_p, ew_p, zero_blk)
  return _final_call(accp, xs2, degp, bg[3], Wh, bhr)


# speedup vs baseline: 2.4293x; 1.2007x over previous
import functools
import jax
import jax.numpy as jnp
from jax import lax
from jax.experimental import pallas as pl
from jax.experimental.pallas import tpu as pltpu
from jax.experimental.pallas import tpu_sc as plsc

N = 10000
HID = 128
OUT = 64
HALF = HID // 2
NC, NS, LANES = 2, 16, 16
STRIPE = 640
NPAD = NS * STRIPE
CHUNK = 128
CPT = 160
EPT = CHUNK * CPT
EPAD = EPT * NS
DEG_CPT = CPT // NC
QB = 20
NBLK = CPT // QB
TROWS = N // NS


@functools.lru_cache(maxsize=None)
def _sc_kernels():
  mesh = plsc.VectorSubcoreMesh(core_axis_name="c", subcore_axis_name="s",
                                num_cores=NC, num_subcores=NS)
  params = pltpu.CompilerParams(use_tc_tiling_on_sc=False)

  @functools.partial(
      pl.kernel,
      out_type=jax.ShapeDtypeStruct((NC, NPAD), jnp.float32),
      mesh=mesh,
      compiler_params=params,
      scratch_types=[
          pltpu.VMEM((DEG_CPT, CHUNK), jnp.int32),
          pltpu.VMEM((DEG_CPT, CHUNK), jnp.float32),
          pltpu.VMEM((STRIPE,), jnp.float32),
          pltpu.VMEM_SHARED((NPAD,), jnp.float32),
      ],
  )
  def deg_kernel(col_hbm, ew_hbm, out_hbm, col_v, ew_v, zeros_v, deg_sh):
    c = lax.axis_index("c")
    s = lax.axis_index("s")
    pltpu.sync_copy(col_hbm.at[s, pl.ds(c * DEG_CPT, DEG_CPT)], col_v)
    pltpu.sync_copy(ew_hbm.at[s, pl.ds(c * DEG_CPT, DEG_CPT)], ew_v)

    @pl.loop(0, STRIPE // LANES)
    def _(j):
      zeros_v[pl.ds(j * LANES, LANES)] = jnp.zeros((LANES,), jnp.float32)

    pltpu.sync_copy(zeros_v, deg_sh.at[pl.ds(s * STRIPE, STRIPE)])
    plsc.subcore_barrier()

    @pl.loop(0, DEG_CPT)
    def _(k):
      pltpu.sync_copy(ew_v.at[k], deg_sh.at[col_v.at[k]], add=True)

    plsc.subcore_barrier()
    pltpu.sync_copy(deg_sh.at[pl.ds(s * STRIPE, STRIPE)],
                    out_hbm.at[c, pl.ds(s * STRIPE, STRIPE)])

  @functools.partial(
      pl.kernel,
      out_type=jax.ShapeDtypeStruct((NC, NPAD, HALF), jnp.float32),
      mesh=mesh,
      compiler_params=params,
      scratch_types=[
          pltpu.VMEM((QB, CHUNK), jnp.int32),
          pltpu.VMEM((QB, CHUNK), jnp.int32),
          pltpu.VMEM((QB, CHUNK), jnp.int32),
          pltpu.VMEM((QB, CHUNK), jnp.int32),
          pltpu.VMEM((QB, CHUNK), jnp.float32),
          pltpu.VMEM((QB, CHUNK), jnp.float32),
          pltpu.VMEM((CHUNK, HALF), jnp.float32),
          pltpu.VMEM((CHUNK, HALF), jnp.float32),
          pltpu.VMEM((CHUNK, HALF), jnp.float32),
          pltpu.VMEM((CHUNK, HALF), jnp.float32),
          pltpu.VMEM_SHARED((N, HALF), jnp.float32),
          pltpu.VMEM_SHARED((NPAD, HALF), jnp.float32),
          pltpu.SemaphoreType.DMA,
          pltpu.SemaphoreType.DMA,
          pltpu.SemaphoreType.DMA,
          pltpu.SemaphoreType.DMA,
          pltpu.SemaphoreType.DMA,
          pltpu.SemaphoreType.DMA,
      ],
  )
  def agg_kernel(xs_hbm, row_hbm, col_hbm, ew_hbm, zero_hbm, out_hbm,
                 rowq0, rowq1, colq0, colq1, ewq0, ewq1, gbuf0, gbuf1,
                 sbuf0, sbuf1, table_sh, acc_sh,
                 gsem0, gsem1, ssem0, ssem1, msem0, msem1):
    c = lax.axis_index("c")
    s = lax.axis_index("s")
    pltpu.sync_copy(xs_hbm.at[pl.ds(c * N + s * TROWS, TROWS)],
                    table_sh.at[pl.ds(s * TROWS, TROWS)])
    pltpu.sync_copy(zero_hbm, acc_sh.at[pl.ds(s * STRIPE, STRIPE)])
    pltpu.sync_copy(row_hbm.at[s, pl.ds(0, QB)], rowq0)
    pltpu.sync_copy(col_hbm.at[s, pl.ds(0, QB)], colq0)
    pltpu.sync_copy(ew_hbm.at[s, pl.ds(0, QB)], ewq0)
    plsc.subcore_barrier()

    bufs = ((rowq0, colq0, ewq0, msem0), (rowq1, colq1, ewq1, msem1))

    def process(k, gbuf, gsem, sbuf, ssem, rq, cq, eq, first):
      pltpu.make_async_copy(table_sh.at[rq.at[k]], gbuf, gsem).wait()

      def wait_scat():
        pltpu.make_async_copy(sbuf, acc_sh.at[cq.at[k]], ssem).wait()
      if first:
        @pl.when(k >= 2)
        def _():
          wait_scat()
      else:
        wait_scat()

      @plsc.parallel_loop(0, CHUNK // LANES)
      def _(jg):
        wv = eq[k, pl.ds(jg * LANES, LANES)]
        for jj in range(LANES):
          w = wv[jj]
          j = jg * LANES + jj
          for g in range(HALF // LANES):
            sl = pl.ds(g * LANES, LANES)
            sbuf[j, sl] = gbuf[j, sl] * w

      @pl.when(k + 2 < QB)
      def _():
        pltpu.async_copy(table_sh.at[rq.at[k + 2]], gbuf, gsem)

      pltpu.async_copy(sbuf, acc_sh.at[cq.at[k]], ssem, add=True)

    for b in range(NBLK):
      rq, cq, eq, _ = bufs[b % 2]
      nrq, ncq, neq, nmsem = bufs[(b + 1) % 2]
      if b + 1 < NBLK:
        off = (b + 1) * QB
        pltpu.async_copy(row_hbm.at[s, pl.ds(off, QB)], nrq, nmsem)
        pltpu.async_copy(col_hbm.at[s, pl.ds(off, QB)], ncq, nmsem)
        pltpu.async_copy(ew_hbm.at[s, pl.ds(off, QB)], neq, nmsem)

      pltpu.async_copy(table_sh.at[rq.at[0]], gbuf0, gsem0)
      pltpu.async_copy(table_sh.at[rq.at[1]], gbuf1, gsem1)

      @pl.loop(0, QB, step=2)
      def _(k):
        process(k, gbuf0, gsem0, sbuf0, ssem0, rq, cq, eq, b == 0)
        process(k + 1, gbuf1, gsem1, sbuf1, ssem1, rq, cq, eq, b == 0)

      if b + 1 < NBLK:
        off = (b + 1) * QB
        pltpu.make_async_copy(row_hbm.at[s, pl.ds(off, QB)], nrq, nmsem).wait()
        pltpu.make_async_copy(col_hbm.at[s, pl.ds(off, QB)], ncq, nmsem).wait()
        pltpu.make_async_copy(ew_hbm.at[s, pl.ds(off, QB)], neq, nmsem).wait()

    pltpu.make_async_copy(sbuf0, acc_sh.at[colq1.at[0]], ssem0).wait()
    pltpu.make_async_copy(sbuf1, acc_sh.at[colq1.at[1]], ssem1).wait()

    plsc.subcore_barrier()
    pltpu.sync_copy(acc_sh.at[pl.ds(s * STRIPE, STRIPE)],
                    out_hbm.at[c, pl.ds(s * STRIPE, STRIPE), :])

  return deg_kernel, agg_kernel


BM = 2000
GRID = N // BM


def _elu(v):
  return jnp.where(v > 0, v, jnp.exp(jnp.minimum(v, 0.0)) - 1.0)


def _dis_of(degp_ref):
  deg = degp_ref[:, 0] + degp_ref[:, 1] + 2.0
  return lax.rsqrt(deg)[:, None]


def _split2(xs, ref):
  ref[0] = xs[:, :HALF]
  ref[1] = xs[:, HALF:]


def _cat2(ref):
  return jnp.concatenate([ref[0], ref[1]], axis=-1)


def _mlp_body(x_ref, degp_ref, w1, b1, w2, b2, w3, b3, wg, xs2_ref):
  h = _elu(jnp.dot(x_ref[:], w1[:], preferred_element_type=jnp.float32) + b1[:])
  h = _elu(jnp.dot(h, w2[:], preferred_element_type=jnp.float32) + b2[:])
  h = _elu(jnp.dot(h, w3[:], preferred_element_type=jnp.float32) + b3[:])
  xs = _dis_of(degp_ref) * jnp.dot(h, wg[:], preferred_element_type=jnp.float32)
  _split2(xs, xs2_ref)


def _combine_body(accp_ref, xs2_ref, degp_ref, b_ref, wg, out2_ref):
  dis = _dis_of(degp_ref)
  agg = _cat2(accp_ref) + 2.0 * _cat2(xs2_ref)
  h = _elu(dis * agg + b_ref[:])
  _split2(dis * jnp.dot(h, wg[:], preferred_element_type=jnp.float32), out2_ref)


def _final_body(accp_ref, xs2_ref, degp_ref, b_ref, wh, bh, out_ref):
  dis = _dis_of(degp_ref)
  agg = _cat2(accp_ref) + 2.0 * _cat2(xs2_ref)
  h = _elu(dis * agg + b_ref[:])
  out_ref[:] = jnp.dot(h, wh[:], preferred_element_type=jnp.float32) + bh[:]


def _full_spec(shape):
  return pl.BlockSpec(shape, lambda i: tuple(0 for _ in shape))


_degp_spec = pl.BlockSpec((BM, 2), lambda i: (i, 0))
_half2_spec = pl.BlockSpec((2, BM, HALF), lambda i: (0, i, 0))
_xs_out_specs = _half2_spec


def _xs_out_shapes():
  return jax.ShapeDtypeStruct((2, N, HALF), jnp.float32)


def _mlp_call(x, degp, W1, b1, W2, b2, W3, b3, Wg1):
  return pl.pallas_call(
      _mlp_body,
      grid=(GRID,),
      in_specs=[pl.BlockSpec((BM, HID), lambda i: (i, 0)), _degp_spec,
                _full_spec((HID, HID)), _full_spec((1, HID)),
                _full_spec((HID, HID)), _full_spec((1, HID)),
                _full_spec((HID, HID)), _full_spec((1, HID)),
                _full_spec((HID, HID))],
      out_specs=_xs_out_specs,
      out_shape=_xs_out_shapes(),
  )(x, degp, W1, b1, W2, b2, W3, b3, Wg1)


def _combine_call(accp, xs2, degp, b, Wg_next):
  return pl.pallas_call(
      _combine_body,
      grid=(GRID,),
      in_specs=[_half2_spec, _half2_spec, _degp_spec,
                _full_spec((1, HID)), _full_spec((HID, HID))],
      out_specs=_xs_out_specs,
      out_shape=_xs_out_shapes(),
  )(accp, xs2, degp, b, Wg_next)


def _final_call(accp, xs2, degp, b, Wh, bh):
  return pl.pallas_call(
      _final_body,
      grid=(GRID,),
      in_specs=[_half2_spec, _half2_spec, _degp_spec,
                _full_spec((1, HID)), _full_spec((HID, OUT)),
                _full_spec((1, OUT))],
      out_specs=pl.BlockSpec((BM, OUT), lambda i: (i, 0)),
      out_shape=jax.ShapeDtypeStruct((N, OUT), jnp.float32),
  )(accp, xs2, degp, b, Wh, bh)


def kernel(x, edge_index, edge_weight, W1, b1, W2, b2, W3, b3,
           Wg1, bg1, Wg2, bg2, Wg3, bg3, Wg4, bg4, Wh, bh):
  deg_kernel, agg_kernel = _sc_kernels()
  E = edge_weight.shape[0]
  pad = EPAD - E
  row = edge_index[0].astype(jnp.int32)
  col = edge_index[1].astype(jnp.int32)
  row_p = jnp.concatenate([row, jnp.zeros((pad,), jnp.int32)]).reshape(NS, CPT, CHUNK)
  col_p = jnp.concatenate([col, jnp.zeros((pad,), jnp.int32)]).reshape(NS, CPT, CHUNK)
  ew_p = jnp.concatenate(
      [edge_weight.astype(jnp.float32), jnp.zeros((pad,), jnp.float32)]
  ).reshape(NS, CPT, CHUNK)
  zero_blk = jnp.zeros((STRIPE, HALF), jnp.float32)

  b1r, b2r, b3r = b1.reshape(1, HID), b2.reshape(1, HID), b3.reshape(1, HID)
  bg = [b.reshape(1, HID) for b in (bg1, bg2, bg3, bg4)]
  bhr = bh.reshape(1, OUT)

  degp = deg_kernel(col_p, ew_p).T
  xs2 = _mlp_call(x, degp, W1, b1r, W2, b2r, W3, b3r, Wg1)
  for Wn, bl in ((Wg2, bg[0]), (Wg3, bg[1]), (Wg4, bg[2])):
    accp = agg_kernel(xs2.reshape(2 * N, HALF), row_p, col_p, ew_p, zero_blk)
    xs2 = _combine_call(accp, xs2, degp, bl, Wn)
  accp = agg_kernel(xs2.reshape(2 * N, HALF), row_p, col_p, ew_p, zero_blk)
  return _final_call(accp, xs2, degp, bg[3], Wh, bhr)
